# baseline (speedup 1.0000x reference)
import functools

import jax
import jax.numpy as jnp
from jax import lax
from jax.experimental import pallas as pl
from jax.experimental.pallas import tpu as pltpu
from jax.experimental.pallas import tpu_sc as plsc

BATCH = 32
SEQ = 2048
D = 128
NUM_WORKERS = 32
CHUNK = SEQ // NUM_WORKERS
VECS = D // 16
GROUP = 4
ROWS = GROUP * CHUNK
NSTEP = BATCH // GROUP
BLK = 8
NBLK = BATCH // BLK


@functools.partial(
    pl.kernel,
    mesh=plsc.VectorSubcoreMesh(core_axis_name="c", subcore_axis_name="s"),
    out_type=jax.ShapeDtypeStruct((BATCH, SEQ, D), jnp.float32),
    scratch_types=[
        pltpu.VMEM((2 * BLK, SEQ), jnp.int32),
        pltpu.VMEM((CHUNK, D), jnp.float32),
        pltpu.VMEM((ROWS, D), jnp.float32),
        pltpu.VMEM((ROWS, D), jnp.float32),
        pltpu.SemaphoreType.DMA,
        pltpu.SemaphoreType.DMA,
        pltpu.SemaphoreType.DMA,
        pltpu.SemaphoreType.DMA,
        pltpu.SemaphoreType.DMA,
    ],
)
def _emb_kernel(idx_hbm, table_hbm, pos_hbm, out_hbm,
                blk_v, pos_v, rows0, rows1,
                gsem0, gsem1, wsem0, wsem1, isem):
    c = lax.axis_index("c")
    s = lax.axis_index("s")
    wid = s * 2 + c
    base = wid * CHUNK

    rows_v = (rows0, rows1)
    gsem = (gsem0, gsem1)
    wsem = (wsem0, wsem1)

    def load_blk(k, sync=False):
        src_off = pl.multiple_of(k * BLK, BLK)
        dst_off = pl.multiple_of((k % 2) * BLK, BLK)
        src = idx_hbm.at[pl.ds(src_off, BLK), :]
        dst = blk_v.at[pl.ds(dst_off, BLK), :]
        if sync:
            pltpu.sync_copy(src, dst)
        else:
            pltpu.async_copy(src, dst, isem)

    def wait_blk():
        pltpu.make_async_copy(idx_hbm.at[pl.ds(0, BLK), :],
                              blk_v.at[pl.ds(0, BLK), :], isem).wait()

    def idx_ref(g, j):
        b = g * GROUP + j
        return blk_v.at[b % (2 * BLK), pl.ds(base, CHUNK)]

    def start_gathers(g, p):
        for j in range(GROUP):
            pltpu.async_copy(table_hbm.at[idx_ref(g, j)],
                             rows_v[p].at[pl.ds(j * CHUNK, CHUNK), :], gsem[p])

    def wait_gathers(g, p):
        for j in range(GROUP):
            pltpu.make_async_copy(
                table_hbm.at[idx_ref(g, j)],
                rows_v[p].at[pl.ds(j * CHUNK, CHUNK), :], gsem[p]).wait()

    def start_writes(g, p):
        for j in range(GROUP):
            pltpu.async_copy(
                rows_v[p].at[pl.ds(j * CHUNK, CHUNK), :],
                out_hbm.at[g * GROUP + j, pl.ds(base, CHUNK), :], wsem[p])

    def wait_writes(p):
        pltpu.make_async_copy(
            rows_v[p], out_hbm.at[0, pl.ds(0, SEQ), :].at[pl.ds(0, ROWS), :],
            wsem[p]).wait()

    def add_pos(p):
        rows = rows_v[p]

        def add_body(r2, carry):
            for u in range(2):
                r = r2 * 2 + u
                for cc in range(VECS):
                    sl = pl.ds(cc * 16, 16)
                    pv = pos_v[r, sl]
                    for j in range(GROUP):
                        plsc.addupdate(rows.at[j * CHUNK + r, sl], pv)
            return carry

        lax.fori_loop(0, CHUNK // 2, add_body, 0)

    pltpu.sync_copy(pos_hbm.at[0, pl.ds(base, CHUNK), :], pos_v)
    load_blk(0, sync=True)
    load_blk(1)
    start_gathers(0, 0)

    def group_body(h, carry):
        @pl.when(h >= 1)
        def _():
            wait_writes(1)
        start_gathers(2 * h + 1, 1)
        wait_gathers(2 * h, 0)
        add_pos(0)
        start_writes(2 * h, 0)

        @pl.when(h < NSTEP // 2 - 1)
        def _():
            wait_blk()
            wait_writes(0)
            start_gathers(2 * h + 2, 0)
        wait_gathers(2 * h + 1, 1)
        @pl.when(h < NBLK - 2)
        def _():
            load_blk(h + 2)
        add_pos(1)
        start_writes(2 * h + 1, 1)
        return carry

    lax.fori_loop(0, NSTEP // 2, group_body, 0)

    for p in range(2):
        wait_writes(p)


def kernel(input_ids, emb_table, pos_enc):
    return _emb_kernel(input_ids.astype(jnp.int32), emb_table, pos_enc)


# speedup vs baseline: 1.2316x; 1.2316x over previous
import functools

import jax
import jax.numpy as jnp
from jax import lax
from jax.experimental import pallas as pl
from jax.experimental.pallas import tpu as pltpu
from jax.experimental.pallas import tpu_sc as plsc

BATCH = 32
SEQ = 2048
D = 128
NUM_WORKERS = 32
CHUNK = SEQ // NUM_WORKERS
VECS = D // 16
GROUP = 2
ROWS = GROUP * CHUNK
NSTEP = BATCH // GROUP
NBUF = 4
UNROLL = 4


@functools.partial(
    pl.kernel,
    mesh=plsc.VectorSubcoreMesh(core_axis_name="c", subcore_axis_name="s"),
    out_type=jax.ShapeDtypeStruct((BATCH, SEQ, D), jnp.float32),
    scratch_types=(
        [pltpu.VMEM((1, ROWS), jnp.int32) for _ in range(NBUF)]
        + [pltpu.VMEM((CHUNK, D), jnp.float32)]
        + [pltpu.VMEM((ROWS, D), jnp.float32) for _ in range(NBUF)]
        + [pltpu.SemaphoreType.DMA for _ in range(3 * NBUF)]
    ),
)
def _emb_kernel(idx_hbm, table_hbm, pos_hbm, out_hbm, *refs):
    idx_v = refs[0:NBUF]
    pos_v = refs[NBUF]
    rows_v = refs[NBUF + 1:2 * NBUF + 1]
    gsem = refs[2 * NBUF + 1:3 * NBUF + 1]
    wsem = refs[3 * NBUF + 1:4 * NBUF + 1]
    isem = refs[4 * NBUF + 1:5 * NBUF + 1]

    c = lax.axis_index("c")
    s = lax.axis_index("s")
    wid = s * 2 + c
    base = wid * CHUNK

    def load_idx(g, p, sync=False):
        for j in range(GROUP):
            off = pl.multiple_of((g * GROUP + j) * SEQ + base, CHUNK)
            dst = idx_v[p].at[0, pl.ds(j * CHUNK, CHUNK)]
            if sync:
                pltpu.sync_copy(idx_hbm.at[pl.ds(off, CHUNK)], dst)
            else:
                pltpu.async_copy(idx_hbm.at[pl.ds(off, CHUNK)], dst, isem[p])

    def wait_idx(p):
        for j in range(GROUP):
            pltpu.make_async_copy(
                idx_hbm.at[pl.ds(0, CHUNK)],
                idx_v[p].at[0, pl.ds(0, CHUNK)], isem[p]).wait()

    def start_gather(p):
        pltpu.async_copy(table_hbm.at[idx_v[p].at[0]], rows_v[p], gsem[p])

    def wait_gather(p):
        pltpu.make_async_copy(
            table_hbm.at[idx_v[p].at[0]], rows_v[p], gsem[p]).wait()

    def start_writes(g, p):
        for j in range(GROUP):
            pltpu.async_copy(
                rows_v[p].at[pl.ds(j * CHUNK, CHUNK), :],
                out_hbm.at[g * GROUP + j, pl.ds(base, CHUNK), :], wsem[p])

    def wait_writes(p):
        pltpu.make_async_copy(
            rows_v[p], out_hbm.at[0, pl.ds(0, SEQ), :].at[pl.ds(0, ROWS), :],
            wsem[p]).wait()

    def add_pos(p):
        rows = rows_v[p]

        def add_body(r2, carry):
            for u in range(2):
                r = r2 * 2 + u
                for cc in range(VECS):
                    sl = pl.ds(cc * 16, 16)
                    pv = pos_v[r, sl]
                    for j in range(GROUP):
                        plsc.addupdate(rows.at[j * CHUNK + r, sl], pv)
            return carry

        lax.fori_loop(0, CHUNK // 2, add_body, 0)

    pltpu.sync_copy(pos_hbm.at[0, pl.ds(base, CHUNK), :], pos_v)
    load_idx(0, 0, sync=True)
    start_gather(0)
    load_idx(1, 1)

    def step(g, p, h):
        np_ = (p + 1) % NBUF

        def launch_next():
            wait_idx(np_)
            if p == NBUF - 1:
                wait_writes(np_)
            else:
                @pl.when(h >= 1)
                def _():
                    wait_writes(np_)
            start_gather(np_)

        if p == UNROLL - 1:
            @pl.when(h < NSTEP // UNROLL - 1)
            def _():
                launch_next()
        else:
            launch_next()

        wait_gather(p)

        pf = (p + 2) % NBUF

        def prefetch_idx():
            load_idx(g + 2, pf)

        if p >= UNROLL - 2:
            @pl.when(h < NSTEP // UNROLL - 1)
            def _():
                prefetch_idx()
        else:
            prefetch_idx()

        add_pos(p)
        start_writes(g, p)

    def group_body(h, carry):
        for p in range(UNROLL):
            step(h * UNROLL + p, p, h)
        return carry

    lax.fori_loop(0, NSTEP // UNROLL, group_body, 0)

    for p in range(NBUF):
        wait_writes(p)


def kernel(input_ids, emb_table, pos_enc):
    return _emb_kernel(input_ids.astype(jnp.int32).reshape(BATCH * SEQ),
                       emb_table, pos_enc)
